# baseline (speedup 1.0000x reference)
import jax
import jax.numpy as jnp
from jax import lax
from jax.experimental import pallas as pl
from jax.experimental.pallas import tpu as pltpu
from jax.experimental.pallas import tpu_sc as plsc

_NB = 16
_N = 4096
_NCH = _N // 16
_CUTS = (1023, 2047, 3071)


def _sc_body(verts, out, cols, ubuf, h1, h2, h3, keybuf, offs, outv):
    cid = lax.axis_index("c")
    sid = lax.axis_index("s")
    wid = sid * 2 + cid

    @pl.when(wid < _NB)
    def _():
        zeros16 = jnp.zeros((16,), jnp.int32)

        cnt0, _ = plsc.scan_count(zeros16)
        base0 = jnp.min(cnt0)
        ones16 = jnp.full((16,), 1, jnp.int32)
        basec = jnp.max(plsc.cumsum(ones16)) - 15

        def exclc(x):
            return plsc.cumsum(x) - x * basec

        def popc(m):
            return plsc.all_reduce_population_count(m)

        def zero_fill(ref, nch):
            def zb(i, _):
                for u in range(8):
                    ref[pl.ds((i * 8 + u) * 16, 16)] = zeros16
                return 0
            lax.fori_loop(0, nch // 8, zb, 0)

        def hist_walk(ref, rbase, targets, maxch):
            nt = len(targets)
            rmax = targets[-1]
            def cond(st):
                return st[1] <= rmax
            def body(st):
                j, tot = st[0], st[1]
                caps = list(st[2:])
                h = ref[pl.ds(rbase + j * 16, 16)]
                ex = exclc(h) + tot
                ntot = tot + jnp.sum(h)
                for k in range(nt):
                    r = targets[k]
                    hit = (tot <= r) & (r < ntot)
                    mask = ex <= r
                    tl = j * 16 + jnp.max(popc(mask)) - 1
                    Ll = jnp.max(jnp.where(mask, ex, -1))
                    caps[2 * k] = jnp.where(hit, tl, caps[2 * k])
                    caps[2 * k + 1] = jnp.where(hit, Ll, caps[2 * k + 1])
                return (j + 1, ntot, *caps)
            init = (jnp.int32(0), jnp.int32(0)) + (jnp.int32(0),) * (2 * nt)
            st = lax.while_loop(cond, body, init)
            return [(st[2 + 2 * k], st[3 + 2 * k]) for k in range(nt)]

        pltpu.sync_copy(verts.at[wid], cols)

        q_all = [[None] * 3 for _ in range(3)]
        m3_all = [[None] * 3 for _ in range(3)]

        for a in range(3):
            zero_fill(h1, 128)
            zero_fill(h2, 384)
            zero_fill(h3, 192)

            def p1(i, _):
                x = cols[a, pl.ds(i * 16, 16)]
                bits = plsc.bitcast(x, jnp.int32)
                skey = bits ^ ((bits >> 31) & jnp.int32(0x7FFFFFFF))
                skey = jnp.where(skey == -1, 0, skey)
                ubuf[a, pl.ds(i * 16, 16)] = skey
                b1 = ((skey >> 21) & 2047) ^ 1024
                cnt, last = plsc.scan_count(b1)
                plsc.addupdate_scatter(h1, [b1], cnt - base0 + 1, mask=last)
                return 0
            lax.fori_loop(0, _NCH, p1, 0, unroll=4)

            caps1 = hist_walk(h1, jnp.int32(0), [jnp.int32(r) for r in _CUTS], 128)
            t1s = [c[0] for c in caps1]
            m1s = [jnp.int32(_CUTS[k]) - caps1[k][1] for k in range(3)]

            reg_b = jnp.where(t1s[1] != t1s[0], 1, 0)
            reg_c = reg_b + jnp.where(t1s[2] != t1s[1], 1, 0)
            regs2 = [jnp.int32(0), reg_b, reg_c]

            def p2(i, _):
                skey = ubuf[a, pl.ds(i * 16, 16)]
                b1 = ((skey >> 21) & 2047) ^ 1024
                b2 = (skey >> 10) & 2047
                m0 = b1 == t1s[0]
                m1 = b1 == t1s[1]
                m2 = b1 == t1s[2]
                ridx = jnp.where(m0, regs2[0], jnp.where(m1, regs2[1], regs2[2]))
                anym = m0 | m1 | m2
                idx = b2 + ridx * 2048
                cnt, last = plsc.scan_count(idx, anym)
                plsc.addupdate_scatter(h2, [idx], cnt - base0 + 1, mask=last & anym)
                return 0
            lax.fori_loop(0, _NCH, p2, 0, unroll=4)

            t2s = []
            m2s = []
            for k in range(3):
                (t2, L2), = hist_walk(h2, regs2[k] * 2048, [m1s[k]], 128)
                t2s.append(t2)
                m2s.append(m1s[k] - L2)

            pref22 = [((t1s[k] ^ 1024) << 11) | t2s[k] for k in range(3)]
            reg_b3 = jnp.where(pref22[1] != pref22[0], 1, 0)
            reg_c3 = reg_b3 + jnp.where(pref22[2] != pref22[1], 1, 0)
            regs3 = [jnp.int32(0), reg_b3, reg_c3]

            def p3(i, _):
                skey = ubuf[a, pl.ds(i * 16, 16)]
                hi22 = (skey >> 10) & jnp.int32(0x3FFFFF)
                b3v = skey & 1023
                m0 = hi22 == pref22[0]
                m1 = hi22 == pref22[1]
                m2 = hi22 == pref22[2]
                ridx = jnp.where(m0, regs3[0], jnp.where(m1, regs3[1], regs3[2]))
                anym = m0 | m1 | m2
                idx = b3v + ridx * 1024
                cnt, last = plsc.scan_count(idx, anym)
                plsc.addupdate_scatter(h3, [idx], cnt - base0 + 1, mask=last & anym)
                return 0
            lax.fori_loop(0, _NCH, p3, 0, unroll=4)

            for k in range(3):
                (t3, L3), = hist_walk(h3, regs3[k] * 1024, [m2s[k]], 64)
                m3_all[a][k] = m2s[k] - L3
                q_all[a][k] = (pref22[k] << 10) | t3

        for u in range(4):
            offs[pl.ds(u * 16, 16)] = zeros16

        def pc(i, carry):
            carry = list(carry)
            key = zeros16
            for a in range(3):
                skey = ubuf[a, pl.ds(i * 16, 16)]
                binv = zeros16
                for k in range(3):
                    q = q_all[a][k]
                    m3 = m3_all[a][k]
                    lt = skey < q
                    eqm = skey == q
                    pre = exclc(eqm.astype(jnp.int32)) + carry[a * 3 + k]
                    lower = lt | (eqm & (pre <= m3))
                    binv = binv + (1 - lower.astype(jnp.int32))
                    carry[a * 3 + k] = carry[a * 3 + k] + popc(eqm)
                key = key + binv * (1, 4, 16)[a]
            keybuf[pl.ds(i * 16, 16)] = key
            cnt, last = plsc.scan_count(key)
            plsc.addupdate_scatter(offs, [key], cnt - base0 + 1, mask=last)
            return tuple(carry)
        lax.fori_loop(0, _NCH, pc, (zeros16,) * 9, unroll=2)

        def oc(i, tot):
            h = offs[pl.ds(i * 16, 16)]
            offs[pl.ds(i * 16, 16)] = exclc(h) + tot
            return tot + jnp.sum(h)
        lax.fori_loop(0, 4, oc, jnp.int32(0), unroll=4)

        def pf(i, _):
            key = keybuf[pl.ds(i * 16, 16)]
            cnt, last = plsc.scan_count(key)
            cz = cnt - base0
            basev = plsc.load_gather(offs, [key])
            pos3 = (basev + cz) * 3
            plsc.addupdate_scatter(offs, [key], cz + 1, mask=last)
            plsc.store_scatter(outv, [pos3], cols[0, pl.ds(i * 16, 16)])
            plsc.store_scatter(outv, [pos3 + 1], cols[1, pl.ds(i * 16, 16)])
            plsc.store_scatter(outv, [pos3 + 2], cols[2, pl.ds(i * 16, 16)])
            return 0
        lax.fori_loop(0, _NCH, pf, 0, unroll=2)

        pltpu.sync_copy(outv, out.at[wid])


def kernel(vertices):
    verts_t = vertices.transpose(0, 2, 1)
    f = pl.kernel(
        _sc_body,
        out_type=jax.ShapeDtypeStruct((_NB, _N * 3), jnp.float32),
        compiler_params=pltpu.CompilerParams(needs_layout_passes=False),
        mesh=plsc.VectorSubcoreMesh(
            core_axis_name="c", subcore_axis_name="s",
            num_cores=2, num_subcores=16),
        scratch_types=[
            pltpu.VMEM((3, _N), jnp.float32),
            pltpu.VMEM((3, _N), jnp.int32),
            pltpu.VMEM((2048,), jnp.int32),
            pltpu.VMEM((6144,), jnp.int32),
            pltpu.VMEM((3072,), jnp.int32),
            pltpu.VMEM((_N,), jnp.int32),
            pltpu.VMEM((64,), jnp.int32),
            pltpu.VMEM((_N * 3,), jnp.float32),
        ],
    )
    out = f(verts_t)
    return out.reshape(_NB, 64, 192)


# speedup vs baseline: 1.2246x; 1.2246x over previous
import jax
import jax.numpy as jnp
from jax import lax
from jax.experimental import pallas as pl
from jax.experimental.pallas import tpu as pltpu
from jax.experimental.pallas import tpu_sc as plsc

_NB = 16
_N = 4096
_NCH = _N // 16
_CUTS = (1023, 2047, 3071)


def _sc_body(verts, out, cols, ubuf, h1, h2, h3, keybuf, offs, outv):
    cid = lax.axis_index("c")
    sid = lax.axis_index("s")
    wid = sid * 2 + cid

    @pl.when(wid < _NB)
    def _():
        zeros16 = jnp.zeros((16,), jnp.int32)

        cnt0, _ = plsc.scan_count(zeros16)
        base0 = jnp.min(cnt0)
        ones16 = jnp.full((16,), 1, jnp.int32)
        basec = jnp.max(plsc.cumsum(ones16)) - 15

        def exclc(x):
            return plsc.cumsum(x) - x * basec

        def popc(m):
            return plsc.all_reduce_population_count(m)

        def zero_fill(ref, nch):
            def zb(i, _):
                for u in range(8):
                    ref[pl.ds((i * 8 + u) * 16, 16)] = zeros16
                return 0
            lax.fori_loop(0, nch // 8, zb, 0)

        def hist_scan(ref, rbase, targets, nch):
            nt = len(targets)
            neg1 = jnp.full((16,), -1, jnp.int32)
            def body(i, st):
                tot = st[0]
                accs = list(st[1:1 + nt])
                lmaxs = list(st[1 + nt:])
                h = ref[pl.ds(rbase + i * 16, 16)]
                ex = exclc(h) + tot
                for k in range(nt):
                    mask = ex <= targets[k]
                    accs[k] = accs[k] + popc(mask)
                    lmaxs[k] = jnp.maximum(lmaxs[k], jnp.where(mask, ex, neg1))
                return (tot + jnp.sum(h), *accs, *lmaxs)
            init = (jnp.int32(0),) + (zeros16,) * nt + (neg1,) * nt
            st = lax.fori_loop(0, nch, body, init, unroll=2)
            return [(jnp.max(st[1 + k]) - 1, jnp.max(st[1 + nt + k]))
                    for k in range(nt)]

        pltpu.sync_copy(verts.at[wid], cols)

        q_all = [[None] * 3 for _ in range(3)]
        m3_all = [[None] * 3 for _ in range(3)]

        for a in range(3):
            zero_fill(h1, 128)
            zero_fill(h2, 384)
            zero_fill(h3, 192)

            def p1(i, _):
                x = cols[a, pl.ds(i * 16, 16)]
                bits = plsc.bitcast(x, jnp.int32)
                skey = bits ^ ((bits >> 31) & jnp.int32(0x7FFFFFFF))
                skey = jnp.where(skey == -1, 0, skey)
                ubuf[a, pl.ds(i * 16, 16)] = skey
                b1 = ((skey >> 21) & 2047) ^ 1024
                cnt, last = plsc.scan_count(b1)
                plsc.addupdate_scatter(h1, [b1], cnt - base0 + 1, mask=last)
                return 0
            lax.fori_loop(0, _NCH, p1, 0, unroll=2)

            caps1 = hist_scan(h1, jnp.int32(0), [jnp.int32(r) for r in _CUTS], 128)
            t1s = [c[0] for c in caps1]
            m1s = [jnp.int32(_CUTS[k]) - caps1[k][1] for k in range(3)]

            reg_b = jnp.where(t1s[1] != t1s[0], 1, 0)
            reg_c = reg_b + jnp.where(t1s[2] != t1s[1], 1, 0)
            regs2 = [jnp.int32(0), reg_b, reg_c]

            def p2(i, _):
                skey = ubuf[a, pl.ds(i * 16, 16)]
                b1 = ((skey >> 21) & 2047) ^ 1024
                b2 = (skey >> 10) & 2047
                m0 = b1 == t1s[0]
                m1 = b1 == t1s[1]
                m2 = b1 == t1s[2]
                ridx = jnp.where(m0, regs2[0], jnp.where(m1, regs2[1], regs2[2]))
                anym = m0 | m1 | m2
                idx = b2 + ridx * 2048
                cnt, last = plsc.scan_count(idx, anym)
                plsc.addupdate_scatter(h2, [idx], cnt - base0 + 1, mask=last & anym)
                return 0
            lax.fori_loop(0, _NCH, p2, 0, unroll=2)

            t2s = []
            m2s = []
            for k in range(3):
                (t2, L2), = hist_scan(h2, regs2[k] * 2048, [m1s[k]], 128)
                t2s.append(t2)
                m2s.append(m1s[k] - L2)

            pref22 = [((t1s[k] ^ 1024) << 11) | t2s[k] for k in range(3)]
            reg_b3 = jnp.where(pref22[1] != pref22[0], 1, 0)
            reg_c3 = reg_b3 + jnp.where(pref22[2] != pref22[1], 1, 0)
            regs3 = [jnp.int32(0), reg_b3, reg_c3]

            def p3(i, _):
                skey = ubuf[a, pl.ds(i * 16, 16)]
                hi22 = (skey >> 10) & jnp.int32(0x3FFFFF)
                b3v = skey & 1023
                m0 = hi22 == pref22[0]
                m1 = hi22 == pref22[1]
                m2 = hi22 == pref22[2]
                ridx = jnp.where(m0, regs3[0], jnp.where(m1, regs3[1], regs3[2]))
                anym = m0 | m1 | m2
                idx = b3v + ridx * 1024
                cnt, last = plsc.scan_count(idx, anym)
                plsc.addupdate_scatter(h3, [idx], cnt - base0 + 1, mask=last & anym)
                return 0
            lax.fori_loop(0, _NCH, p3, 0, unroll=2)

            for k in range(3):
                (t3, L3), = hist_scan(h3, regs3[k] * 1024, [m2s[k]], 64)
                m3_all[a][k] = m2s[k] - L3
                q_all[a][k] = (pref22[k] << 10) | t3

        for u in range(4):
            offs[pl.ds(u * 16, 16)] = zeros16

        def pc(i, carry):
            carry = list(carry)
            key = zeros16
            for a in range(3):
                skey = ubuf[a, pl.ds(i * 16, 16)]
                binv = zeros16
                for k in range(3):
                    q = q_all[a][k]
                    m3 = m3_all[a][k]
                    lt = skey < q
                    eqm = skey == q
                    pre = exclc(eqm.astype(jnp.int32)) + carry[a * 3 + k]
                    lower = lt | (eqm & (pre <= m3))
                    binv = binv + (1 - lower.astype(jnp.int32))
                    carry[a * 3 + k] = carry[a * 3 + k] + popc(eqm)
                key = key + binv * (1, 4, 16)[a]
            keybuf[pl.ds(i * 16, 16)] = key
            cnt, last = plsc.scan_count(key)
            plsc.addupdate_scatter(offs, [key], cnt - base0 + 1, mask=last)
            return tuple(carry)
        lax.fori_loop(0, _NCH, pc, (zeros16,) * 9, unroll=2)

        def oc(i, tot):
            h = offs[pl.ds(i * 16, 16)]
            offs[pl.ds(i * 16, 16)] = exclc(h) + tot
            return tot + jnp.sum(h)
        lax.fori_loop(0, 4, oc, jnp.int32(0), unroll=4)

        def pf(i, _):
            key = keybuf[pl.ds(i * 16, 16)]
            cnt, last = plsc.scan_count(key)
            cz = cnt - base0
            basev = plsc.load_gather(offs, [key])
            pos3 = (basev + cz) * 3
            plsc.addupdate_scatter(offs, [key], cz + 1, mask=last)
            plsc.store_scatter(outv, [pos3], cols[0, pl.ds(i * 16, 16)])
            plsc.store_scatter(outv, [pos3 + 1], cols[1, pl.ds(i * 16, 16)])
            plsc.store_scatter(outv, [pos3 + 2], cols[2, pl.ds(i * 16, 16)])
            return 0
        lax.fori_loop(0, _NCH, pf, 0, unroll=2)

        pltpu.sync_copy(outv, out.at[wid])


def kernel(vertices):
    verts_t = vertices.transpose(0, 2, 1)
    f = pl.kernel(
        _sc_body,
        out_type=jax.ShapeDtypeStruct((_NB, _N * 3), jnp.float32),
        compiler_params=pltpu.CompilerParams(needs_layout_passes=False),
        mesh=plsc.VectorSubcoreMesh(
            core_axis_name="c", subcore_axis_name="s",
            num_cores=2, num_subcores=16),
        scratch_types=[
            pltpu.VMEM((3, _N), jnp.float32),
            pltpu.VMEM((3, _N), jnp.int32),
            pltpu.VMEM((2048,), jnp.int32),
            pltpu.VMEM((6144,), jnp.int32),
            pltpu.VMEM((3072,), jnp.int32),
            pltpu.VMEM((_N,), jnp.int32),
            pltpu.VMEM((64,), jnp.int32),
            pltpu.VMEM((_N * 3,), jnp.float32),
        ],
    )
    out = f(verts_t)
    return out.reshape(_NB, 64, 192)


# speedup vs baseline: 1.9932x; 1.6276x over previous
import jax
import jax.numpy as jnp
from jax import lax
from jax.experimental import pallas as pl
from jax.experimental.pallas import tpu as pltpu
from jax.experimental.pallas import tpu_sc as plsc

_NB = 16
_N = 4096
_NCH = _N // 16
_CUTS = (1023, 2047, 3071)


def _sc_body(verts, out, cols, ubuf, h1, h2, h3, keybuf, offs, outv):
    cid = lax.axis_index("c")
    sid = lax.axis_index("s")
    wid = sid * 2 + cid

    @pl.when(wid < _NB)
    def _():
        zeros16 = jnp.zeros((16,), jnp.int32)

        cnt0, _ = plsc.scan_count(zeros16)
        base0 = jnp.min(cnt0)
        ones16 = jnp.full((16,), 1, jnp.int32)
        basec = jnp.max(plsc.cumsum(ones16)) - 15

        def exclc(x):
            return plsc.cumsum(x) - x * basec

        def popc(m):
            return plsc.all_reduce_population_count(m)

        def zero_fill(ref, nch):
            def zb(i, _):
                for u in range(8):
                    ref[pl.ds((i * 8 + u) * 16, 16)] = zeros16
                return 0
            lax.fori_loop(0, nch // 8, zb, 0)

        def hist_scan(ref, rbase, targets, nch):
            nt = len(targets)
            neg1 = jnp.full((16,), -1, jnp.int32)
            def body(i, st):
                tot = st[0]
                accs = list(st[1:1 + nt])
                lmaxs = list(st[1 + nt:])
                h = ref[pl.ds(rbase + i * 16, 16)]
                ex = exclc(h) + tot
                for k in range(nt):
                    mask = ex <= targets[k]
                    accs[k] = accs[k] + popc(mask)
                    lmaxs[k] = jnp.maximum(lmaxs[k], jnp.where(mask, ex, neg1))
                return (tot + jnp.sum(h), *accs, *lmaxs)
            init = (jnp.int32(0),) + (zeros16,) * nt + (neg1,) * nt
            st = lax.fori_loop(0, nch, body, init, unroll=2)
            return [(jnp.max(st[1 + k]) - 1, jnp.max(st[1 + nt + k]))
                    for k in range(nt)]

        pltpu.sync_copy(verts.at[wid], cols)

        q_all = [[None] * 3 for _ in range(3)]
        m3_all = [[None] * 3 for _ in range(3)]

        for a in range(3):
            zero_fill(h1, 128)
            zero_fill(h2, 384)
            zero_fill(h3, 192)

            def p1(i):
                x = cols[a, pl.ds(i * 16, 16)]
                bits = plsc.bitcast(x, jnp.int32)
                skey = bits ^ ((bits >> 31) & jnp.int32(0x7FFFFFFF))
                skey = jnp.where(skey == -1, 0, skey)
                ubuf[a, pl.ds(i * 16, 16)] = skey
                b1 = ((skey >> 21) & 2047) ^ 1024
                cnt, last = plsc.scan_count(b1)
                plsc.addupdate_scatter(h1, [b1], cnt - base0 + 1, mask=last)
            plsc.parallel_loop(0, _NCH, unroll=4)(p1)

            caps1 = hist_scan(h1, jnp.int32(0), [jnp.int32(r) for r in _CUTS], 128)
            t1s = [c[0] for c in caps1]
            m1s = [jnp.int32(_CUTS[k]) - caps1[k][1] for k in range(3)]

            reg_b = jnp.where(t1s[1] != t1s[0], 1, 0)
            reg_c = reg_b + jnp.where(t1s[2] != t1s[1], 1, 0)
            regs2 = [jnp.int32(0), reg_b, reg_c]

            def p2(i):
                skey = ubuf[a, pl.ds(i * 16, 16)]
                b1 = ((skey >> 21) & 2047) ^ 1024
                b2 = (skey >> 10) & 2047
                m0 = b1 == t1s[0]
                m1 = b1 == t1s[1]
                m2 = b1 == t1s[2]
                ridx = jnp.where(m0, regs2[0], jnp.where(m1, regs2[1], regs2[2]))
                anym = m0 | m1 | m2
                idx = b2 + ridx * 2048
                cnt, last = plsc.scan_count(idx, anym)
                plsc.addupdate_scatter(h2, [idx], cnt - base0 + 1, mask=last & anym)
            plsc.parallel_loop(0, _NCH, unroll=4)(p2)

            t2s = []
            m2s = []
            for k in range(3):
                (t2, L2), = hist_scan(h2, regs2[k] * 2048, [m1s[k]], 128)
                t2s.append(t2)
                m2s.append(m1s[k] - L2)

            pref22 = [((t1s[k] ^ 1024) << 11) | t2s[k] for k in range(3)]
            reg_b3 = jnp.where(pref22[1] != pref22[0], 1, 0)
            reg_c3 = reg_b3 + jnp.where(pref22[2] != pref22[1], 1, 0)
            regs3 = [jnp.int32(0), reg_b3, reg_c3]

            def p3(i):
                skey = ubuf[a, pl.ds(i * 16, 16)]
                hi22 = (skey >> 10) & jnp.int32(0x3FFFFF)
                b3v = skey & 1023
                m0 = hi22 == pref22[0]
                m1 = hi22 == pref22[1]
                m2 = hi22 == pref22[2]
                ridx = jnp.where(m0, regs3[0], jnp.where(m1, regs3[1], regs3[2]))
                anym = m0 | m1 | m2
                idx = b3v + ridx * 1024
                cnt, last = plsc.scan_count(idx, anym)
                plsc.addupdate_scatter(h3, [idx], cnt - base0 + 1, mask=last & anym)
            plsc.parallel_loop(0, _NCH, unroll=4)(p3)

            for k in range(3):
                (t3, L3), = hist_scan(h3, regs3[k] * 1024, [m2s[k]], 64)
                m3_all[a][k] = m2s[k] - L3
                q_all[a][k] = (pref22[k] << 10) | t3

        for u in range(4):
            offs[pl.ds(u * 16, 16)] = zeros16

        def pc(i, carry):
            carry = list(carry)
            key = zeros16
            for a in range(3):
                skey = ubuf[a, pl.ds(i * 16, 16)]
                binv = zeros16
                for k in range(3):
                    q = q_all[a][k]
                    m3 = m3_all[a][k]
                    lt = skey < q
                    eqm = skey == q
                    pre = exclc(eqm.astype(jnp.int32)) + carry[a * 3 + k]
                    lower = lt | (eqm & (pre <= m3))
                    binv = binv + (1 - lower.astype(jnp.int32))
                    carry[a * 3 + k] = carry[a * 3 + k] + popc(eqm)
                key = key + binv * (1, 4, 16)[a]
            keybuf[pl.ds(i * 16, 16)] = key
            cnt, last = plsc.scan_count(key)
            plsc.addupdate_scatter(offs, [key], cnt - base0 + 1, mask=last)
            return tuple(carry)
        lax.fori_loop(0, _NCH, pc, (zeros16,) * 9, unroll=2)

        def oc(i, tot):
            h = offs[pl.ds(i * 16, 16)]
            offs[pl.ds(i * 16, 16)] = exclc(h) + tot
            return tot + jnp.sum(h)
        lax.fori_loop(0, 4, oc, jnp.int32(0), unroll=4)

        def pf(i, _):
            key = keybuf[pl.ds(i * 16, 16)]
            cnt, last = plsc.scan_count(key)
            cz = cnt - base0
            basev = plsc.load_gather(offs, [key])
            pos3 = (basev + cz) * 3
            plsc.addupdate_scatter(offs, [key], cz + 1, mask=last)
            plsc.store_scatter(outv, [pos3], cols[0, pl.ds(i * 16, 16)])
            plsc.store_scatter(outv, [pos3 + 1], cols[1, pl.ds(i * 16, 16)])
            plsc.store_scatter(outv, [pos3 + 2], cols[2, pl.ds(i * 16, 16)])
            return 0
        lax.fori_loop(0, _NCH, pf, 0, unroll=2)

        pltpu.sync_copy(outv, out.at[wid])


def kernel(vertices):
    verts_t = vertices.transpose(0, 2, 1)
    f = pl.kernel(
        _sc_body,
        out_type=jax.ShapeDtypeStruct((_NB, _N * 3), jnp.float32),
        compiler_params=pltpu.CompilerParams(needs_layout_passes=False),
        mesh=plsc.VectorSubcoreMesh(
            core_axis_name="c", subcore_axis_name="s",
            num_cores=2, num_subcores=16),
        scratch_types=[
            pltpu.VMEM((3, _N), jnp.float32),
            pltpu.VMEM((3, _N), jnp.int32),
            pltpu.VMEM((2048,), jnp.int32),
            pltpu.VMEM((6144,), jnp.int32),
            pltpu.VMEM((3072,), jnp.int32),
            pltpu.VMEM((_N,), jnp.int32),
            pltpu.VMEM((64,), jnp.int32),
            pltpu.VMEM((_N * 3,), jnp.float32),
        ],
    )
    out = f(verts_t)
    return out.reshape(_NB, 64, 192)
